# Initial kernel scaffold; baseline (speedup 1.0000x reference)
import jax
import jax.numpy as jnp
from jax.experimental import pallas as pl


def kernel(disp, atom_node, edge_index):
    raise NotImplementedError("write your pallas kernel here")


# speedup vs baseline: 2.7153x; 2.7153x over previous
import functools

import jax
import jax.numpy as jnp
from jax import lax
from jax.experimental import pallas as pl
from jax.experimental.pallas import tpu as pltpu
from jax.experimental.pallas import tpu_sc as plsc

N_NODES = 100000
N_EDGES = 6400000

NC = 2
NS = 16
NW = NC * NS
EW = N_EDGES // NW
G = 64
CHUNK = 1600
NG = CHUNK // G
NCHUNK = EW // CHUNK
PT = 6256
N_PAD = PT * NS
DRN = PT // 4


def _sc_scatter(disp8, dst_idx, src_idx, zeros):
    mesh = plsc.VectorSubcoreMesh(core_axis_name="c", subcore_axis_name="s")

    @functools.partial(
        pl.kernel,
        out_type=jax.ShapeDtypeStruct((4 * N_PAD, 8), jnp.float32),
        mesh=mesh,
        compiler_params=pltpu.CompilerParams(use_tc_tiling_on_sc=False),
        scratch_types=[
            pltpu.VMEM((CHUNK, 8), jnp.float32),
            pltpu.VMEM((NG, G), jnp.int32),
            pltpu.VMEM((DRN, 8), jnp.float32),
            pltpu.VMEM_SHARED((N_PAD, 8), jnp.float32),
        ],
    )
    def k(disp_hbm, di_hbm, si_hbm, z_hbm, out_hbm, dbuf, sbuf, zbuf, acc):
        cid = lax.axis_index("c")
        sid = lax.axis_index("s")
        wid = sid * NC + cid
        row0 = sid * PT
        base = wid * EW
        gbase = wid * (EW // G)

        for p, idx_hbm in enumerate([di_hbm, si_hbm]):
            pltpu.sync_copy(z_hbm, zbuf)
            for j in range(PT // DRN):
                pltpu.sync_copy(zbuf, acc.at[pl.ds(row0 + j * DRN, DRN)])
            plsc.subcore_barrier()

            def body(i, carry, idx_hbm=idx_hbm):
                pltpu.sync_copy(disp_hbm.at[pl.ds(base + i * CHUNK, CHUNK)], dbuf)
                pltpu.sync_copy(idx_hbm.at[pl.ds(gbase + i * NG, NG)], sbuf)

                def grp(j, c2):
                    pltpu.sync_copy(dbuf.at[pl.ds(j * G, G)],
                                    acc.at[sbuf.at[j]], add=True)
                    return c2

                return lax.fori_loop(0, NG, grp, carry)

            lax.fori_loop(0, NCHUNK, body, 0)
            plsc.subcore_barrier()

            for j in range(PT // DRN):
                r = row0 + j * DRN
                pltpu.sync_copy(acc.at[pl.ds(r, DRN)], zbuf)
                pltpu.sync_copy(
                    zbuf, out_hbm.at[pl.ds((2 * p + cid) * N_PAD + r, DRN)])
            plsc.subcore_barrier()

    return k(disp8, dst_idx, src_idx, zeros)


def kernel(disp, atom_node, edge_index):
    del atom_node
    disp8 = jnp.pad(disp, ((0, 0), (0, 5)))
    idx = edge_index.astype(jnp.int32)
    si = idx[0].reshape(N_EDGES // G, G)
    di = idx[1].reshape(N_EDGES // G, G)
    zeros = jnp.zeros((DRN, 8), jnp.float32)
    out = _sc_scatter(disp8, di, si, zeros).reshape(2, 2, N_PAD, 8)
    res = (out[0, 0] + out[0, 1]) - (out[1, 0] + out[1, 1])
    return res[:N_NODES, :3]


# speedup vs baseline: 2.8108x; 1.0352x over previous
import functools

import jax
import jax.numpy as jnp
from jax import lax
from jax.experimental import pallas as pl
from jax.experimental.pallas import tpu as pltpu
from jax.experimental.pallas import tpu_sc as plsc

N_NODES = 100000
N_EDGES = 6400000

NC = 2
NS = 16
NW = NC * NS
EW = N_EDGES // NW
G = 80
CHUNK = 1600
NG = CHUNK // G
NCHUNK = EW // CHUNK
PT = 6256
N_PAD = PT * NS
DRN = PT // 4


def _sc_scatter(disp8, dst_idx, src_idx, zeros):
    mesh = plsc.VectorSubcoreMesh(core_axis_name="c", subcore_axis_name="s")

    @functools.partial(
        pl.kernel,
        out_type=jax.ShapeDtypeStruct((4 * N_PAD, 8), jnp.float32),
        mesh=mesh,
        compiler_params=pltpu.CompilerParams(use_tc_tiling_on_sc=False),
        scratch_types=[
            pltpu.VMEM((CHUNK, 8), jnp.float32),
            pltpu.VMEM((NG, G), jnp.int32),
            pltpu.VMEM((DRN, 8), jnp.float32),
            pltpu.VMEM_SHARED((N_PAD, 8), jnp.float32),
            pltpu.SemaphoreType.DMA,
        ],
    )
    def k(disp_hbm, di_hbm, si_hbm, z_hbm, out_hbm,
          dbuf, sbuf, zbuf, acc, ssem):
        cid = lax.axis_index("c")
        sid = lax.axis_index("s")
        wid = sid * NC + cid
        row0 = sid * PT
        base = wid * EW
        gbase = wid * (EW // G)

        for p, idx_hbm in enumerate([di_hbm, si_hbm]):
            pltpu.sync_copy(z_hbm, zbuf)
            for j in range(PT // DRN):
                pltpu.sync_copy(zbuf, acc.at[pl.ds(row0 + j * DRN, DRN)])
            plsc.subcore_barrier()

            def body(i, carry, idx_hbm=idx_hbm):
                pltpu.sync_copy(disp_hbm.at[pl.ds(base + i * CHUNK, CHUNK)], dbuf)
                pltpu.sync_copy(idx_hbm.at[pl.ds(gbase + i * NG, NG)], sbuf)
                descs = [
                    pltpu.async_copy(dbuf.at[pl.ds(j * G, G)],
                                     acc.at[sbuf.at[j]], ssem, add=True)
                    for j in range(NG)
                ]
                for d in descs:
                    d.wait()
                return carry

            lax.fori_loop(0, NCHUNK, body, 0)
            plsc.subcore_barrier()

            for j in range(PT // DRN):
                r = row0 + j * DRN
                pltpu.sync_copy(acc.at[pl.ds(r, DRN)], zbuf)
                pltpu.sync_copy(
                    zbuf, out_hbm.at[pl.ds((2 * p + cid) * N_PAD + r, DRN)])
            plsc.subcore_barrier()

    return k(disp8, dst_idx, src_idx, zeros)


def kernel(disp, atom_node, edge_index):
    del atom_node
    disp8 = jnp.pad(disp, ((0, 0), (0, 5)))
    idx = edge_index.astype(jnp.int32)
    si = idx[0].reshape(N_EDGES // G, G)
    di = idx[1].reshape(N_EDGES // G, G)
    zeros = jnp.zeros((DRN, 8), jnp.float32)
    out = _sc_scatter(disp8, di, si, zeros).reshape(2, 2, N_PAD, 8)
    res = (out[0, 0] + out[0, 1]) - (out[1, 0] + out[1, 1])
    return res[:N_NODES, :3]


# speedup vs baseline: 3.2770x; 1.1658x over previous
import functools

import jax
import jax.numpy as jnp
from jax import lax
from jax.experimental import pallas as pl
from jax.experimental.pallas import tpu as pltpu
from jax.experimental.pallas import tpu_sc as plsc

N_NODES = 100000
N_EDGES = 6400000

NC = 2
NS = 16
NW = NC * NS
EW = N_EDGES // NW
G = 80
CHUNK = 1600
NG = CHUNK // G
NCHUNK = EW // CHUNK
NB = CHUNK // 16
PT = 6256
N_PAD = PT * NS
DRN = PT // 4


def _sc_scatter(disp_flat, dst_idx, src_idx, zeros):
    mesh = plsc.VectorSubcoreMesh(core_axis_name="c", subcore_axis_name="s")

    @functools.partial(
        pl.kernel,
        out_type=jax.ShapeDtypeStruct((4 * N_PAD, 8), jnp.float32),
        mesh=mesh,
        compiler_params=pltpu.CompilerParams(use_tc_tiling_on_sc=False, needs_layout_passes=False),
        scratch_types=[
            pltpu.VMEM((3 * CHUNK,), jnp.float32),
            pltpu.VMEM((CHUNK, 8), jnp.float32),
            pltpu.VMEM((NG, G), jnp.int32),
            pltpu.VMEM((DRN, 8), jnp.float32),
            pltpu.VMEM_SHARED((N_PAD, 8), jnp.float32),
            pltpu.SemaphoreType.DMA,
        ],
    )
    def k(disp_hbm, di_hbm, si_hbm, z_hbm, out_hbm,
          fbuf, dbuf, sbuf, zbuf, acc, ssem):
        cid = lax.axis_index("c")
        sid = lax.axis_index("s")
        wid = sid * NC + cid
        row0 = sid * PT
        base = wid * EW
        gbase = wid * (EW // G)

        lanes = lax.iota(jnp.int32, 16)
        rowoff, coloff = [], []
        for v in range(3):
            pvec = lanes + (v * 16)
            q = lax.shift_right_logical(pvec * 21846, 16)
            rowoff.append(q)
            coloff.append(pvec - q * 3)

        pltpu.sync_copy(z_hbm, zbuf)
        pltpu.sync_copy(z_hbm, dbuf.at[pl.ds(0, DRN)])
        pltpu.sync_copy(z_hbm, dbuf.at[pl.ds(36, DRN)])

        for p, idx_hbm in enumerate([di_hbm, si_hbm]):
            for j in range(PT // DRN):
                pltpu.sync_copy(zbuf, acc.at[pl.ds(row0 + j * DRN, DRN)])
            plsc.subcore_barrier()

            def body(i, carry, idx_hbm=idx_hbm):
                pltpu.sync_copy(
                    disp_hbm.at[pl.ds(3 * (base + i * CHUNK), 3 * CHUNK)], fbuf)
                pltpu.sync_copy(idx_hbm.at[pl.ds(gbase + i * NG, NG)], sbuf)

                def repack(b, c2):
                    eb = b * 16
                    for v in range(3):
                        vals = fbuf[pl.ds(b * 48 + v * 16, 16)]
                        plsc.store_scatter(
                            dbuf, [rowoff[v] + eb, coloff[v]], vals)
                    return c2

                lax.fori_loop(0, NB, repack, 0)

                descs = [
                    pltpu.async_copy(dbuf.at[pl.ds(j * G, G)],
                                     acc.at[sbuf.at[j]], ssem, add=True)
                    for j in range(NG)
                ]
                for d in descs:
                    d.wait()
                return carry

            lax.fori_loop(0, NCHUNK, body, 0)
            plsc.subcore_barrier()

            for j in range(PT // DRN):
                r = row0 + j * DRN
                pltpu.sync_copy(acc.at[pl.ds(r, DRN)], zbuf)
                pltpu.sync_copy(
                    zbuf, out_hbm.at[pl.ds((2 * p + cid) * N_PAD + r, DRN)])
            plsc.subcore_barrier()
            pltpu.sync_copy(z_hbm, zbuf)

    return k(disp_flat, dst_idx, src_idx, zeros)


def kernel(disp, atom_node, edge_index):
    del atom_node
    disp_flat = disp.reshape(-1)
    idx = edge_index.astype(jnp.int32)
    si = idx[0].reshape(N_EDGES // G, G)
    di = idx[1].reshape(N_EDGES // G, G)
    zeros = jnp.zeros((DRN, 8), jnp.float32)
    out = _sc_scatter(disp_flat, di, si, zeros).reshape(2, 2, N_PAD, 8)
    res = (out[0, 0] + out[0, 1]) - (out[1, 0] + out[1, 1])
    return res[:N_NODES, :3]
